# baseline (speedup 1.0000x reference)
import functools

import jax
import jax.numpy as jnp
from jax import lax
from jax.experimental import pallas as pl
from jax.experimental.pallas import tpu as pltpu
from jax.experimental.pallas import tpu_sc as plsc

B = 4096
F = 26
D = 32
N_ATN = 16
VOCAB = 1000000
PACK = 4
YROWS = VOCAB // PACK
NCHUNK = 4

_info = plsc.get_sparse_core_info()
_NC, _NS, _L = _info.num_cores, _info.num_subcores, _info.num_lanes
NW = _NC * _NS
BPW = B // NW
CB = BPW // NCHUNK

NBLK = VOCAB // (PACK * D)
NB = 1
NG = NBLK // NB
NG_BASE = NG // NW
NG_REM = NG - NG_BASE * NW
GW = PACK * D * NB
GSZ = D * GW
YG = D * NB
TAIL = VOCAB - NBLK * (PACK * D)
TAIL_Y = TAIL // PACK

_mesh = plsc.VectorSubcoreMesh(core_axis_name="c", subcore_axis_name="s")
_params = pltpu.CompilerParams(needs_layout_passes=False)


@functools.partial(
    pl.kernel,
    mesh=_mesh,
    out_type=jax.ShapeDtypeStruct((YROWS, PACK * D), jnp.float32),
    scratch_types=[
        pltpu.VMEM((4, D, PACK * D), jnp.float32),
        pltpu.VMEM((4, YG, PACK * D), jnp.float32),
        pltpu.VMEM((D * TAIL,), jnp.float32),
        pltpu.VMEM((PACK * D // _L, _L), jnp.int32),
        pltpu.SemaphoreType.DMA,
        pltpu.SemaphoreType.DMA,
    ],
    compiler_params=_params,
)
def _sc_repack(tt_hbm, tail_hbm, gidx_hbm, y_hbm,
               xin_v, yout_v, tail_v, gidx_v, sem_i, sem_o):
    wid = lax.axis_index("s") * _NC + lax.axis_index("c")
    ng_w = NG_BASE + jnp.where(wid < NG_REM, 1, 0)
    pltpu.sync_copy(gidx_hbm, gidx_v)
    ivs = [gidx_v[g, :] for g in range(PACK * D // _L)]

    BW = PACK * D
    rvs = [iv >> 7 for iv in ivs]
    cvs = [iv & (BW - 1) for iv in ivs]

    def grp_of(i):
        return jnp.minimum(wid + NW * i, NG - 1)

    DEPTH = 4

    def start_in(i):
        off = pl.multiple_of(grp_of(i) * GW, GW)
        pltpu.async_copy(tt_hbm.at[:, pl.ds(off, GW)],
                         xin_v.at[i % DEPTH], sem_i)

    def drain_in():
        pltpu.make_async_copy(
            tt_hbm.at[:, pl.ds(0, GW)], xin_v.at[0], sem_i).wait()

    def drain_out():
        pltpu.make_async_copy(
            y_hbm.at[pl.ds(0, YG)], yout_v.at[0], sem_o).wait()

    for p in range(DEPTH - 1):
        start_in(p)

    def body(i, carry):
        ib = i % DEPTH
        drain_in()
        start_in(i + DEPTH - 1)
        @pl.when(i >= DEPTH)
        def _():
            drain_out()
        ibv = jnp.full((_L,), ib, jnp.int32)
        for k in range(D):
            sk = jnp.full((_L,), PACK * k, jnp.int32)
            for g in range(BW // _L):
                val = plsc.load_gather(xin_v, [ibv, rvs[g], cvs[g] + sk])
                yout_v[ib, k, pl.ds(g * _L, _L)] = val
        yoff = pl.multiple_of(grp_of(i) * YG, 8)
        pltpu.async_copy(yout_v.at[ib], y_hbm.at[pl.ds(yoff, YG)], sem_o)
        return carry

    lax.fori_loop(0, ng_w, body, 0)
    for p in range(DEPTH - 1):
        drain_in()
    for p in range(DEPTH):
        drain_out()

    @pl.when(wid == NW - 1)
    def _():
        pltpu.sync_copy(tail_hbm, tail_v)
        for k in range(TAIL_Y):
            sk = jnp.full((_L,), PACK * k, jnp.int32)
            for g in range(BW // _L):
                ivt = rvs[g] * TAIL + (ivs[g] & (PACK - 1))
                val = plsc.load_gather(tail_v, [ivt + sk])
                yout_v[0, k, pl.ds(g * _L, _L)] = val
        pltpu.sync_copy(yout_v.at[0, pl.ds(0, TAIL_Y)],
                        y_hbm.at[pl.ds(YROWS - TAIL_Y, TAIL_Y)])


@functools.partial(
    pl.kernel,
    mesh=_mesh,
    out_type=jax.ShapeDtypeStruct((B, D), jnp.float32),
    scratch_types=[
        pltpu.VMEM((F, BPW), jnp.int32),
        pltpu.VMEM((BPW * F,), jnp.int32),
        pltpu.VMEM((F, CB, PACK * D), jnp.float32),
        pltpu.VMEM((BPW, D), jnp.float32),
        pltpu.SemaphoreType.DMA,
    ],
    compiler_params=_params,
)
def _sc_gather_sum(sb_hbm, mo_hbm, y_hbm, out_hbm,
                   sb_v, mo_v, rows_v, out_v, sem):
    wid = lax.axis_index("s") * _NC + lax.axis_index("c")
    base = wid * BPW

    pltpu.sync_copy(sb_hbm.at[wid], sb_v)
    pltpu.sync_copy(mo_hbm.at[wid], mo_v)

    lanes4 = PACK * lax.broadcasted_iota(jnp.int32, (_L,), 0)

    for c in range(NCHUNK):
        for f in range(F):
            pltpu.async_copy(
                y_hbm.at[sb_v.at[f, pl.ds(c * CB, CB)]], rows_v.at[f], sem)
        for f in range(F):
            pltpu.make_async_copy(
                y_hbm.at[pl.ds(0, CB)], rows_v.at[f], sem).wait()

        def body(j, carry, c=c):
            jf = jnp.full((_L,), (c * CB + j) * F, jnp.int32)
            jv = jnp.full((_L,), j, jnp.int32)
            mos = [plsc.load_gather(mo_v, [jf + f]) for f in range(F)]

            def pick(f, col):
                return plsc.load_gather(
                    rows_v,
                    [jnp.full((_L,), f, jnp.int32), jv,
                     mos[f] + PACK * col + lanes4])

            for col in (0, _L):
                a0 = pick(0, col) + pick(1, col)
                a1 = pick(2, col) + pick(3, col)
                for f in range(4, F, 2):
                    a0 = a0 + pick(f, col)
                    a1 = a1 + pick(f + 1, col)
                out_v[c * CB + j, pl.ds(col, _L)] = a0 + a1
            return carry

        lax.fori_loop(0, CB, body, 0)

    pltpu.sync_copy(out_v, out_hbm.at[pl.ds(base, BPW)])


def _combine_body(sum_ref, val_ref, atn_ref, w_ref, b_ref, act_ref, out_ref):
    inv2f = 1.0 / (2.0 * F)
    s = jnp.sum(val_ref[...], axis=1, keepdims=True)
    onehot = (atn_ref[...] ==
              lax.broadcasted_iota(jnp.int32, (B, N_ATN), 1)
              ).astype(jnp.float32)
    act = jnp.dot(onehot, act_ref[...],
                  preferred_element_type=jnp.float32)
    out_ref[...] = (sum_ref[...] * inv2f
                    + s * (w_ref[...] * inv2f)
                    + b_ref[...] * 0.5
                    + act)


def kernel(idx_d, val_c, atn_idx, table_d, W_c, b_c, action_table):
    idx32 = idx_d.astype(jnp.int32)
    sb_r = ((idx32 >> 2).T
            .reshape(F, NW, BPW)
            .transpose(1, 0, 2))
    mo_r = (idx32 & 3).reshape(NW, BPW * F)

    tt = table_d.T
    gidx = jnp.asarray(
        [[(PACK * g + j // PACK) * GW + j % PACK for j in range(_L)]
         for g in range(PACK * D // _L)], dtype=jnp.int32)
    y = _sc_repack(tt, tt[:, NBLK * (PACK * D):].reshape(D * TAIL), gidx)
    sums = _sc_gather_sum(sb_r, mo_r, y)

    out = pl.pallas_call(
        _combine_body,
        out_shape=jax.ShapeDtypeStruct((B, D), jnp.float32),
    )(
        sums,
        val_c,
        atn_idx.astype(jnp.int32).reshape(B, 1),
        W_c.astype(jnp.float32).reshape(1, D),
        b_c.reshape(1, D),
        action_table,
    )
    return out


# speedup vs baseline: 2.8530x; 2.8530x over previous
import functools

import jax
import jax.numpy as jnp
from jax import lax
from jax.experimental import pallas as pl
from jax.experimental.pallas import tpu as pltpu
from jax.experimental.pallas import tpu_sc as plsc

B = 4096
F = 26
D = 32
N_ATN = 16
VOCAB = 1000000
PACK = 4
YROWS = VOCAB // PACK
NCHUNK = 4

_info = plsc.get_sparse_core_info()
_NC, _NS, _L = _info.num_cores, _info.num_subcores, _info.num_lanes
NW = _NC * _NS
BPW = B // NW
CB = BPW // NCHUNK

NBLK = VOCAB // (PACK * D)
NB = 1
NG = NBLK // NB
NG_BASE = NG // NW
NG_REM = NG - NG_BASE * NW
GW = PACK * D * NB
GSZ = D * GW
YG = D * NB
TAIL = VOCAB - NBLK * (PACK * D)
TAIL_Y = TAIL // PACK

_mesh = plsc.VectorSubcoreMesh(core_axis_name="c", subcore_axis_name="s")
_params = pltpu.CompilerParams(needs_layout_passes=False)


@functools.partial(
    pl.kernel,
    mesh=_mesh,
    out_type=jax.ShapeDtypeStruct((YROWS, PACK * D), jnp.float32),
    scratch_types=[
        pltpu.VMEM((4, D, PACK * D), jnp.float32),
        pltpu.VMEM((4, YG, PACK * D), jnp.float32),
        pltpu.VMEM((D * TAIL,), jnp.float32),
        pltpu.VMEM((PACK * D // _L, _L), jnp.int32),
        pltpu.SemaphoreType.DMA,
        pltpu.SemaphoreType.DMA,
    ],
    compiler_params=_params,
)
def _sc_repack(tt_hbm, tail_hbm, gidx_hbm, y_hbm,
               xin_v, yout_v, tail_v, gidx_v, sem_i, sem_o):
    wid = lax.axis_index("s") * _NC + lax.axis_index("c")
    ng_w = NG_BASE + jnp.where(wid < NG_REM, 1, 0)
    pltpu.sync_copy(gidx_hbm, gidx_v)
    ivs = [gidx_v[g, :] for g in range(PACK * D // _L)]

    BW = PACK * D
    rvs = [iv >> 7 for iv in ivs]
    cvs = [iv & (BW - 1) for iv in ivs]

    def grp_of(i):
        return jnp.minimum(wid + NW * i, NG - 1)

    DEPTH = 4

    def start_in(i):
        off = pl.multiple_of(grp_of(i) * GW, GW)
        pltpu.async_copy(tt_hbm.at[:, pl.ds(off, GW)],
                         xin_v.at[i % DEPTH], sem_i)

    def drain_in():
        pltpu.make_async_copy(
            tt_hbm.at[:, pl.ds(0, GW)], xin_v.at[0], sem_i).wait()

    def drain_out():
        pltpu.make_async_copy(
            y_hbm.at[pl.ds(0, YG)], yout_v.at[0], sem_o).wait()

    for p in range(DEPTH - 1):
        start_in(p)

    def body(i, carry):
        ib = i % DEPTH
        drain_in()
        start_in(i + DEPTH - 1)
        @pl.when(i >= DEPTH)
        def _():
            drain_out()
        ibv = jnp.full((_L,), ib, jnp.int32)

        @plsc.parallel_loop(0, D, unroll=4)
        def _(k):
            sk = jnp.full((_L,), PACK * k, jnp.int32)
            for g in range(BW // _L):
                val = plsc.load_gather(xin_v, [ibv, rvs[g], cvs[g] + sk])
                yout_v[ib, k, pl.ds(g * _L, _L)] = val
        yoff = pl.multiple_of(grp_of(i) * YG, 8)
        pltpu.async_copy(yout_v.at[ib], y_hbm.at[pl.ds(yoff, YG)], sem_o)
        return carry

    lax.fori_loop(0, ng_w, body, 0)
    for p in range(DEPTH - 1):
        drain_in()
    for p in range(DEPTH):
        drain_out()

    @pl.when(wid == NW - 1)
    def _():
        pltpu.sync_copy(tail_hbm, tail_v)
        for k in range(TAIL_Y):
            sk = jnp.full((_L,), PACK * k, jnp.int32)
            for g in range(BW // _L):
                ivt = rvs[g] * TAIL + (ivs[g] & (PACK - 1))
                val = plsc.load_gather(tail_v, [ivt + sk])
                yout_v[0, k, pl.ds(g * _L, _L)] = val
        pltpu.sync_copy(yout_v.at[0, pl.ds(0, TAIL_Y)],
                        y_hbm.at[pl.ds(YROWS - TAIL_Y, TAIL_Y)])


@functools.partial(
    pl.kernel,
    mesh=_mesh,
    out_type=jax.ShapeDtypeStruct((B, D), jnp.float32),
    scratch_types=[
        pltpu.VMEM((F, BPW), jnp.int32),
        pltpu.VMEM((BPW * F,), jnp.int32),
        pltpu.VMEM((F, CB, PACK * D), jnp.float32),
        pltpu.VMEM((BPW, D), jnp.float32),
        pltpu.SemaphoreType.DMA,
    ],
    compiler_params=_params,
)
def _sc_gather_sum(sb_hbm, mo_hbm, y_hbm, out_hbm,
                   sb_v, mo_v, rows_v, out_v, sem):
    wid = lax.axis_index("s") * _NC + lax.axis_index("c")
    base = wid * BPW

    pltpu.sync_copy(sb_hbm.at[wid], sb_v)
    pltpu.sync_copy(mo_hbm.at[wid], mo_v)

    lanes4 = PACK * lax.broadcasted_iota(jnp.int32, (_L,), 0)

    for c in range(NCHUNK):
        for f in range(F):
            pltpu.async_copy(
                y_hbm.at[sb_v.at[f, pl.ds(c * CB, CB)]], rows_v.at[f], sem)
        for f in range(F):
            pltpu.make_async_copy(
                y_hbm.at[pl.ds(0, CB)], rows_v.at[f], sem).wait()

        def body(j, carry, c=c):
            jf = jnp.full((_L,), (c * CB + j) * F, jnp.int32)
            jv = jnp.full((_L,), j, jnp.int32)
            mos = [plsc.load_gather(mo_v, [jf + f]) for f in range(F)]

            def pick(f, col):
                return plsc.load_gather(
                    rows_v,
                    [jnp.full((_L,), f, jnp.int32), jv,
                     mos[f] + PACK * col + lanes4])

            for col in (0, _L):
                a0 = pick(0, col) + pick(1, col)
                a1 = pick(2, col) + pick(3, col)
                for f in range(4, F, 2):
                    a0 = a0 + pick(f, col)
                    a1 = a1 + pick(f + 1, col)
                out_v[c * CB + j, pl.ds(col, _L)] = a0 + a1
            return carry

        lax.fori_loop(0, CB, body, 0)

    pltpu.sync_copy(out_v, out_hbm.at[pl.ds(base, BPW)])


def _combine_body(sum_ref, val_ref, atn_ref, w_ref, b_ref, act_ref, out_ref):
    inv2f = 1.0 / (2.0 * F)
    s = jnp.sum(val_ref[...], axis=1, keepdims=True)
    onehot = (atn_ref[...] ==
              lax.broadcasted_iota(jnp.int32, (B, N_ATN), 1)
              ).astype(jnp.float32)
    act = jnp.dot(onehot, act_ref[...],
                  preferred_element_type=jnp.float32)
    out_ref[...] = (sum_ref[...] * inv2f
                    + s * (w_ref[...] * inv2f)
                    + b_ref[...] * 0.5
                    + act)


def kernel(idx_d, val_c, atn_idx, table_d, W_c, b_c, action_table):
    idx32 = idx_d.astype(jnp.int32)
    sb_r = ((idx32 >> 2).T
            .reshape(F, NW, BPW)
            .transpose(1, 0, 2))
    mo_r = (idx32 & 3).reshape(NW, BPW * F)

    tt = table_d.T
    gidx = jnp.asarray(
        [[(PACK * g + j // PACK) * GW + j % PACK for j in range(_L)]
         for g in range(PACK * D // _L)], dtype=jnp.int32)
    y = _sc_repack(tt, tt[:, NBLK * (PACK * D):].reshape(D * TAIL), gidx)
    sums = _sc_gather_sum(sb_r, mo_r, y)

    out = pl.pallas_call(
        _combine_body,
        out_shape=jax.ShapeDtypeStruct((B, D), jnp.float32),
    )(
        sums,
        val_c,
        atn_idx.astype(jnp.int32).reshape(B, 1),
        W_c.astype(jnp.float32).reshape(1, D),
        b_c.reshape(1, D),
        action_table,
    )
    return out


# speedup vs baseline: 2.8685x; 1.0054x over previous
import functools

import jax
import jax.numpy as jnp
from jax import lax
from jax.experimental import pallas as pl
from jax.experimental.pallas import tpu as pltpu
from jax.experimental.pallas import tpu_sc as plsc

B = 4096
F = 26
D = 32
N_ATN = 16
VOCAB = 1000000
PACK = 4
YROWS = VOCAB // PACK
NCHUNK = 4

_info = plsc.get_sparse_core_info()
_NC, _NS, _L = _info.num_cores, _info.num_subcores, _info.num_lanes
NW = _NC * _NS
BPW = B // NW
CB = BPW // NCHUNK

NBLK = VOCAB // (PACK * D)
NB = 1
NG = NBLK // NB
NG_BASE = NG // NW
NG_REM = NG - NG_BASE * NW
GW = PACK * D * NB
GSZ = D * GW
YG = D * NB
TAIL = VOCAB - NBLK * (PACK * D)
TAIL_Y = TAIL // PACK

_mesh = plsc.VectorSubcoreMesh(core_axis_name="c", subcore_axis_name="s")
_params = pltpu.CompilerParams(needs_layout_passes=False)


@functools.partial(
    pl.kernel,
    mesh=_mesh,
    out_type=jax.ShapeDtypeStruct((YROWS, PACK * D), jnp.float32),
    scratch_types=[
        pltpu.VMEM((4, D, PACK * D), jnp.float32),
        pltpu.VMEM((4, YG, PACK * D), jnp.float32),
        pltpu.VMEM((D * TAIL,), jnp.float32),
        pltpu.VMEM((PACK * D // _L, _L), jnp.int32),
        pltpu.SemaphoreType.DMA,
        pltpu.SemaphoreType.DMA,
    ],
    compiler_params=_params,
)
def _sc_repack(tt_hbm, tail_hbm, gidx_hbm, y_hbm,
               xin_v, yout_v, tail_v, gidx_v, sem_i, sem_o):
    wid = lax.axis_index("s") * _NC + lax.axis_index("c")
    ng_w = NG_BASE + jnp.where(wid < NG_REM, 1, 0)
    pltpu.sync_copy(gidx_hbm, gidx_v)
    ivs = [gidx_v[g, :] for g in range(PACK * D // _L)]

    BW = PACK * D
    rvs = [iv >> 7 for iv in ivs]
    cvs = [iv & (BW - 1) for iv in ivs]

    def grp_of(i):
        return jnp.minimum(wid + NW * i, NG - 1)

    DEPTH = 4

    def start_in(i):
        off = pl.multiple_of(grp_of(i) * GW, GW)
        pltpu.async_copy(tt_hbm.at[:, pl.ds(off, GW)],
                         xin_v.at[i % DEPTH], sem_i)

    def drain_in():
        pltpu.make_async_copy(
            tt_hbm.at[:, pl.ds(0, GW)], xin_v.at[0], sem_i).wait()

    def drain_out():
        pltpu.make_async_copy(
            y_hbm.at[pl.ds(0, YG)], yout_v.at[0], sem_o).wait()

    for p in range(DEPTH - 1):
        start_in(p)

    def body(i, carry):
        ib = i % DEPTH
        drain_in()
        start_in(i + DEPTH - 1)
        @pl.when(i >= DEPTH)
        def _():
            drain_out()
        ibv = jnp.full((_L,), ib, jnp.int32)

        @plsc.parallel_loop(0, D, unroll=4)
        def _(k):
            sk = jnp.full((_L,), PACK * k, jnp.int32)
            for g in range(BW // _L):
                val = plsc.load_gather(xin_v, [ibv, rvs[g], cvs[g] + sk])
                yout_v[ib, k, pl.ds(g * _L, _L)] = val
        yoff = pl.multiple_of(grp_of(i) * YG, 8)
        pltpu.async_copy(yout_v.at[ib], y_hbm.at[pl.ds(yoff, YG)], sem_o)
        return carry

    lax.fori_loop(0, ng_w, body, 0)
    for p in range(DEPTH - 1):
        drain_in()
    for p in range(DEPTH):
        drain_out()

    @pl.when(wid == NW - 1)
    def _():
        pltpu.sync_copy(tail_hbm, tail_v)
        for k in range(TAIL_Y):
            sk = jnp.full((_L,), PACK * k, jnp.int32)
            for g in range(BW // _L):
                ivt = rvs[g] * TAIL + (ivs[g] & (PACK - 1))
                val = plsc.load_gather(tail_v, [ivt + sk])
                yout_v[0, k, pl.ds(g * _L, _L)] = val
        pltpu.sync_copy(yout_v.at[0, pl.ds(0, TAIL_Y)],
                        y_hbm.at[pl.ds(YROWS - TAIL_Y, TAIL_Y)])


@functools.partial(
    pl.kernel,
    mesh=_mesh,
    out_type=jax.ShapeDtypeStruct((B, D), jnp.float32),
    scratch_types=[
        pltpu.VMEM((F, BPW), jnp.int32),
        pltpu.VMEM((BPW * F,), jnp.int32),
        pltpu.VMEM((F, CB, PACK * D), jnp.float32),
        pltpu.VMEM((BPW, D), jnp.float32),
        pltpu.SemaphoreType.DMA,
    ],
    compiler_params=_params,
)
def _sc_gather_sum(sb_hbm, mo_hbm, y_hbm, out_hbm,
                   sb_v, mo_v, rows_v, out_v, sem):
    wid = lax.axis_index("s") * _NC + lax.axis_index("c")
    base = wid * BPW

    pltpu.sync_copy(sb_hbm.at[wid], sb_v)
    pltpu.sync_copy(mo_hbm.at[wid], mo_v)

    lanes4 = PACK * lax.broadcasted_iota(jnp.int32, (_L,), 0)

    for c in range(NCHUNK):
        for f in range(F):
            pltpu.async_copy(
                y_hbm.at[sb_v.at[f, pl.ds(c * CB, CB)]], rows_v.at[f], sem)
        for f in range(F):
            pltpu.make_async_copy(
                y_hbm.at[pl.ds(0, CB)], rows_v.at[f], sem).wait()

        def body(j, carry, c=c):
            jf = jnp.full((_L,), (c * CB + j) * F, jnp.int32)
            jv = jnp.full((_L,), j, jnp.int32)

            def mo(f):
                return plsc.load_gather(mo_v, [jf + f])

            def pick(m, f, col):
                return plsc.load_gather(
                    rows_v,
                    [jnp.full((_L,), f, jnp.int32), jv,
                     m + PACK * col + lanes4])

            m0, m1 = mo(0), mo(1)
            a0, b0 = pick(m0, 0, 0), pick(m0, 0, _L)
            a1, b1 = pick(m1, 1, 0), pick(m1, 1, _L)
            for f in range(2, F, 2):
                me, mw = mo(f), mo(f + 1)
                a0 = a0 + pick(me, f, 0)
                b0 = b0 + pick(me, f, _L)
                a1 = a1 + pick(mw, f + 1, 0)
                b1 = b1 + pick(mw, f + 1, _L)
            out_v[c * CB + j, pl.ds(0, _L)] = a0 + a1
            out_v[c * CB + j, pl.ds(_L, _L)] = b0 + b1
            return carry

        lax.fori_loop(0, CB, body, 0)

    pltpu.sync_copy(out_v, out_hbm.at[pl.ds(base, BPW)])


def _combine_body(sum_ref, val_ref, atn_ref, w_ref, b_ref, act_ref, out_ref):
    inv2f = 1.0 / (2.0 * F)
    s = jnp.sum(val_ref[...], axis=1, keepdims=True)
    onehot = (atn_ref[...] ==
              lax.broadcasted_iota(jnp.int32, (B, N_ATN), 1)
              ).astype(jnp.float32)
    act = jnp.dot(onehot, act_ref[...],
                  preferred_element_type=jnp.float32)
    out_ref[...] = (sum_ref[...] * inv2f
                    + s * (w_ref[...] * inv2f)
                    + b_ref[...] * 0.5
                    + act)


def kernel(idx_d, val_c, atn_idx, table_d, W_c, b_c, action_table):
    idx32 = idx_d.astype(jnp.int32)
    sb_r = ((idx32 >> 2).T
            .reshape(F, NW, BPW)
            .transpose(1, 0, 2))
    mo_r = (idx32 & 3).reshape(NW, BPW * F)

    tt = table_d.T
    gidx = jnp.asarray(
        [[(PACK * g + j // PACK) * GW + j % PACK for j in range(_L)]
         for g in range(PACK * D // _L)], dtype=jnp.int32)
    y = _sc_repack(tt, tt[:, NBLK * (PACK * D):].reshape(D * TAIL), gidx)
    sums = _sc_gather_sum(sb_r, mo_r, y)

    out = pl.pallas_call(
        _combine_body,
        out_shape=jax.ShapeDtypeStruct((B, D), jnp.float32),
    )(
        sums,
        val_c,
        atn_idx.astype(jnp.int32).reshape(B, 1),
        W_c.astype(jnp.float32).reshape(1, D),
        b_c.reshape(1, D),
        action_table,
    )
    return out
